# baseline (speedup 1.0000x reference)
import functools
import jax
import jax.numpy as jnp
from jax import lax
from jax.experimental import pallas as pl
from jax.experimental.pallas import tpu as pltpu
from jax.experimental.pallas import tpu_sc as plsc

N = 10000
E = 160000
D = 256
H = 128
NC = 2
NS = 16
SLAB = 624
_EXTRA = ((0, 9984), (1, 9992))
CHUNK = 125


def _tile_slabs(s, fn):
    fn(s * SLAB, SLAB)
    for tile, st in _EXTRA:

        @pl.when(s == tile)
        def _():
            fn(st, 8)

H_CH = E // (NC * NS) // CHUNK
S_CH = E // NS // CHUNK

_mesh = lambda: plsc.VectorSubcoreMesh(core_axis_name="c", subcore_axis_name="s")


HW = 128


@functools.partial(
    pl.kernel,
    out_type=jax.ShapeDtypeStruct((NC, N, HW), jnp.float32),
    mesh=_mesh(),
    scratch_types=[
        pltpu.VMEM_SHARED((N, HW), jnp.float32),
        pltpu.VMEM((H_CH, CHUNK), jnp.int32),
        pltpu.VMEM((CHUNK, HW), jnp.float32),
        pltpu.SemaphoreType.DMA,
    ],
)
def _deg_kernel(dst_h, zeros_h, ones_h, hist_out, acc, idx_v, ones_v, sem):
    c = lax.axis_index("c")
    s = lax.axis_index("s")
    w = s * NC + c
    pltpu.sync_copy(ones_h, ones_v)
    pltpu.sync_copy(dst_h.at[w], idx_v)
    _tile_slabs(s, lambda st, sz: pltpu.sync_copy(
        zeros_h.at[pl.ds(st, sz)], acc.at[pl.ds(st, sz)]))
    plsc.subcore_barrier()

    descs = [
        pltpu.async_copy(ones_v, acc.at[idx_v.at[j]], sem, add=True)
        for j in range(H_CH)
    ]
    for d in descs:
        d.wait()
    plsc.subcore_barrier()
    _tile_slabs(s, lambda st, sz: pltpu.sync_copy(
        acc.at[pl.ds(st, sz)], hist_out.at[c, pl.ds(st, sz)]))


@functools.partial(
    pl.kernel,
    out_type=jax.ShapeDtypeStruct((NC, N, H), jnp.float32),
    mesh=_mesh(),
    scratch_types=[
        pltpu.VMEM_SHARED((N, H), jnp.float32),
        pltpu.VMEM((S_CH, CHUNK), jnp.int32),
        pltpu.VMEM((CHUNK,), jnp.int32),
        pltpu.VMEM((CHUNK,), jnp.int32),
        pltpu.VMEM((CHUNK, H), jnp.float32),
        pltpu.VMEM((CHUNK, H), jnp.float32),
        pltpu.SemaphoreType.DMA,
        pltpu.SemaphoreType.DMA,
        pltpu.SemaphoreType.DMA,
        pltpu.SemaphoreType.DMA,
    ],
)
def _agg_kernel(z3, src_h, dst_h, dummy, agg3, acc, src_v, dc0, dc1, r0, r1, sg0, sg1, ss0, ss1):
    c = lax.axis_index("c")
    s = lax.axis_index("s")
    zc = z3.at[c]
    oc = agg3.at[c]
    pltpu.sync_copy(src_h.at[s], src_v)

    _tile_slabs(s, lambda st, sz: pltpu.sync_copy(
        zc.at[pl.ds(st, sz)], acc.at[pl.ds(st, sz)]))
    plsc.subcore_barrier()

    pltpu.async_copy(zc.at[src_v.at[0]], r0, sg0)
    pltpu.async_copy(zc.at[src_v.at[1]], r1, sg1)

    def body(g, carry):
        j0 = 2 * g
        pltpu.sync_copy(dst_h.at[s, j0], dc0)
        pltpu.make_async_copy(dummy, r0, sg0).wait()
        pltpu.async_copy(r0, acc.at[dc0], ss0, add=True)
        pltpu.sync_copy(dst_h.at[s, j0 + 1], dc1)
        pltpu.make_async_copy(dummy, r1, sg1).wait()
        pltpu.async_copy(r1, acc.at[dc1], ss1, add=True)
        pltpu.make_async_copy(r0, dummy, ss0).wait()
        pltpu.async_copy(zc.at[src_v.at[jnp.minimum(j0 + 2, S_CH - 1)]], r0, sg0)
        pltpu.make_async_copy(r1, dummy, ss1).wait()
        pltpu.async_copy(zc.at[src_v.at[jnp.minimum(j0 + 3, S_CH - 1)]], r1, sg1)
        return carry

    lax.fori_loop(0, S_CH // 2, body, 0)
    pltpu.make_async_copy(dummy, r0, sg0).wait()
    pltpu.make_async_copy(dummy, r1, sg1).wait()
    plsc.subcore_barrier()
    _tile_slabs(s, lambda st, sz: pltpu.sync_copy(
        acc.at[pl.ds(st, sz)], oc.at[pl.ds(st, sz)]))


def _mmscale_body(hist_ref, x_ref, w_ref, z3_ref, dinv_ref):
    xw = jnp.dot(x_ref[...], w_ref[...], preferred_element_type=jnp.float32)
    deg = 1.0 + hist_ref[0, :, :1] + hist_ref[1, :, :1]
    dinv = lax.rsqrt(jnp.maximum(deg, 1.0))
    z = dinv * xw
    z3_ref[0] = z[:, :H]
    z3_ref[1] = z[:, H:]
    dinv_ref[...] = dinv


def _bn_body(a3_ref, dinv_ref, b_ref, g_ref, be_ref, o_ref, s1, s2):
    nb = pl.num_programs(0) // 2
    i = pl.program_id(0)
    p = i // nb
    t = jnp.concatenate([a3_ref[0], a3_ref[1]], axis=1)
    t = t * dinv_ref[...] + b_ref[...]

    @pl.when(i == 0)
    def _():
        s1[...] = jnp.zeros_like(s1)
        s2[...] = jnp.zeros_like(s2)

    @pl.when(p == 0)
    def _():
        s1[...] += jnp.sum(t, axis=0, keepdims=True)
        s2[...] += jnp.sum(t * t, axis=0, keepdims=True)

    mean = s1[...] * (1.0 / N)
    var = s2[...] * (1.0 / N) - mean * mean
    rstd = lax.rsqrt(var + 1e-5)
    y = jnp.maximum(g_ref[...] * (t - mean) * rstd + be_ref[...], 0.0)
    o_ref[...] = jnp.where(p == 1, y, t)


def kernel(x, edge_index, batch, W, b, gamma, beta):
    del batch
    src = edge_index[0].astype(jnp.int32)
    dst = edge_index[1].astype(jnp.int32)
    dst_hist = dst.reshape(NC * NS, H_CH, CHUNK)
    src_sc = src.reshape(NS, S_CH, CHUNK)
    dst_sc = dst.reshape(NS, S_CH, CHUNK)
    zeros_h = jnp.zeros((N, HW), jnp.float32)
    ones_h = jnp.ones((CHUNK, HW), jnp.float32)

    hist = _deg_kernel(dst_hist, zeros_h, ones_h)

    BLK = 1000
    nb = N // BLK
    z3, dinv = pl.pallas_call(
        _mmscale_body,
        grid=(nb,),
        in_specs=[
            pl.BlockSpec((NC, BLK, HW), lambda i: (0, i, 0)),
            pl.BlockSpec((BLK, D), lambda i: (i, 0)),
            pl.BlockSpec((D, D), lambda i: (0, 0)),
        ],
        out_specs=[
            pl.BlockSpec((NC, BLK, H), lambda i: (0, i, 0)),
            pl.BlockSpec((BLK, 1), lambda i: (i, 0)),
        ],
        out_shape=[
            jax.ShapeDtypeStruct((NC, N, H), jnp.float32),
            jax.ShapeDtypeStruct((N, 1), jnp.float32),
        ],
    )(hist, x, W)

    agg3 = _agg_kernel(z3, src_sc, dst_sc, jnp.zeros((CHUNK, H), jnp.float32))

    y = pl.pallas_call(
        _bn_body,
        grid=(2 * nb,),
        in_specs=[
            pl.BlockSpec((NC, BLK, H), lambda i: (0, i % nb, 0)),
            pl.BlockSpec((BLK, 1), lambda i: (i % nb, 0)),
            pl.BlockSpec((1, D), lambda i: (0, 0)),
            pl.BlockSpec((1, D), lambda i: (0, 0)),
            pl.BlockSpec((1, D), lambda i: (0, 0)),
        ],
        out_specs=pl.BlockSpec((BLK, D), lambda i: (i % nb, 0)),
        out_shape=jax.ShapeDtypeStruct((N, D), jnp.float32),
        scratch_shapes=[
            pltpu.VMEM((1, D), jnp.float32),
            pltpu.VMEM((1, D), jnp.float32),
        ],
    )(agg3, dinv, b.reshape(1, D), gamma.reshape(1, D), beta.reshape(1, D))
    return y


# speedup vs baseline: 1.0260x; 1.0260x over previous
import functools
import jax
import jax.numpy as jnp
from jax import lax
from jax.experimental import pallas as pl
from jax.experimental.pallas import tpu as pltpu
from jax.experimental.pallas import tpu_sc as plsc

N = 10000
E = 160000
D = 256
H = 128
NC = 2
NS = 16
SLAB = 624
_EXTRA = ((0, 9984), (1, 9992))
CHUNK = 125


def _tile_slabs(s, fn):
    fn(s * SLAB, SLAB)
    for tile, st in _EXTRA:

        @pl.when(s == tile)
        def _():
            fn(st, 8)

H_CH = E // (NC * NS) // CHUNK
S_CH = E // NS // CHUNK

_mesh = lambda: plsc.VectorSubcoreMesh(core_axis_name="c", subcore_axis_name="s")


HW = 128


@functools.partial(
    pl.kernel,
    out_type=jax.ShapeDtypeStruct((NC, N, HW), jnp.float32),
    mesh=_mesh(),
    scratch_types=[
        pltpu.VMEM_SHARED((N, HW), jnp.float32),
        pltpu.VMEM((H_CH, CHUNK), jnp.int32),
        pltpu.VMEM((CHUNK, HW), jnp.float32),
        pltpu.SemaphoreType.DMA,
    ],
)
def _deg_kernel(dst_h, zeros_h, ones_h, hist_out, acc, idx_v, ones_v, sem):
    c = lax.axis_index("c")
    s = lax.axis_index("s")
    w = s * NC + c
    pltpu.sync_copy(ones_h, ones_v)
    pltpu.sync_copy(dst_h.at[w], idx_v)
    _tile_slabs(s, lambda st, sz: pltpu.sync_copy(
        zeros_h.at[pl.ds(st, sz)], acc.at[pl.ds(st, sz)]))
    plsc.subcore_barrier()

    descs = [
        pltpu.async_copy(ones_v, acc.at[idx_v.at[j]], sem, add=True)
        for j in range(H_CH)
    ]
    for d in descs:
        d.wait()
    plsc.subcore_barrier()
    _tile_slabs(s, lambda st, sz: pltpu.sync_copy(
        acc.at[pl.ds(st, sz)], hist_out.at[c, pl.ds(st, sz)]))


@functools.partial(
    pl.kernel,
    out_type=jax.ShapeDtypeStruct((NC, N, H), jnp.float32),
    mesh=_mesh(),
    scratch_types=[
        pltpu.VMEM_SHARED((N, H), jnp.float32),
        pltpu.VMEM((S_CH, CHUNK), jnp.int32),
        pltpu.VMEM((CHUNK,), jnp.int32),
        pltpu.VMEM((CHUNK,), jnp.int32),
        pltpu.VMEM((CHUNK, H), jnp.float32),
        pltpu.VMEM((CHUNK, H), jnp.float32),
        pltpu.SemaphoreType.DMA,
        pltpu.SemaphoreType.DMA,
    ],
)
def _agg_kernel(z3, src_h, dst_h, dummy, agg3, acc, src_v, dc0, dc1, r0, r1, sg0, sg1):
    c = lax.axis_index("c")
    s = lax.axis_index("s")
    zc = z3.at[c]
    oc = agg3.at[c]
    pltpu.sync_copy(src_h.at[s], src_v)

    _tile_slabs(s, lambda st, sz: pltpu.sync_copy(
        zc.at[pl.ds(st, sz)], acc.at[pl.ds(st, sz)]))
    plsc.subcore_barrier()

    pltpu.async_copy(zc.at[src_v.at[0]], r0, sg0)

    def body(g, carry):
        j0 = 2 * g
        j2 = jnp.minimum(j0 + 2, S_CH - 1)
        pltpu.async_copy(zc.at[src_v.at[j0 + 1]], r1, sg1)
        pltpu.sync_copy(dst_h.at[s, j0], dc0)
        pltpu.make_async_copy(dummy, r0, sg0).wait()
        pltpu.sync_copy(r0, acc.at[dc0], add=True)
        pltpu.async_copy(zc.at[src_v.at[j2]], r0, sg0)
        pltpu.sync_copy(dst_h.at[s, j0 + 1], dc1)
        pltpu.make_async_copy(dummy, r1, sg1).wait()
        pltpu.sync_copy(r1, acc.at[dc1], add=True)
        return carry

    lax.fori_loop(0, S_CH // 2, body, 0)
    pltpu.make_async_copy(dummy, r0, sg0).wait()
    plsc.subcore_barrier()
    _tile_slabs(s, lambda st, sz: pltpu.sync_copy(
        acc.at[pl.ds(st, sz)], oc.at[pl.ds(st, sz)]))


def _mmscale_body(hist_ref, x_ref, w_ref, z3_ref, dinv_ref):
    xw = jnp.dot(x_ref[...], w_ref[...], preferred_element_type=jnp.float32)
    deg = 1.0 + hist_ref[0, :, :1] + hist_ref[1, :, :1]
    dinv = lax.rsqrt(jnp.maximum(deg, 1.0))
    z = dinv * xw
    z3_ref[0] = z[:, :H]
    z3_ref[1] = z[:, H:]
    dinv_ref[...] = dinv


def _bn_body(a3_ref, dinv_ref, b_ref, g_ref, be_ref, o_ref, s1, s2):
    nb = pl.num_programs(0) // 2
    i = pl.program_id(0)
    p = i // nb
    t = jnp.concatenate([a3_ref[0], a3_ref[1]], axis=1)
    t = t * dinv_ref[...] + b_ref[...]

    @pl.when(i == 0)
    def _():
        s1[...] = jnp.zeros_like(s1)
        s2[...] = jnp.zeros_like(s2)

    @pl.when(p == 0)
    def _():
        s1[...] += jnp.sum(t, axis=0, keepdims=True)
        s2[...] += jnp.sum(t * t, axis=0, keepdims=True)

    mean = s1[...] * (1.0 / N)
    var = s2[...] * (1.0 / N) - mean * mean
    rstd = lax.rsqrt(var + 1e-5)
    y = jnp.maximum(g_ref[...] * (t - mean) * rstd + be_ref[...], 0.0)
    o_ref[...] = jnp.where(p == 1, y, t)


def kernel(x, edge_index, batch, W, b, gamma, beta):
    del batch
    src = edge_index[0].astype(jnp.int32)
    dst = edge_index[1].astype(jnp.int32)
    dst_hist = dst.reshape(NC * NS, H_CH, CHUNK)
    src_sc = src.reshape(NS, S_CH, CHUNK)
    dst_sc = dst.reshape(NS, S_CH, CHUNK)
    zeros_h = jnp.zeros((N, HW), jnp.float32)
    ones_h = jnp.ones((CHUNK, HW), jnp.float32)

    hist = _deg_kernel(dst_hist, zeros_h, ones_h)

    BLK = 1000
    nb = N // BLK
    z3, dinv = pl.pallas_call(
        _mmscale_body,
        grid=(nb,),
        in_specs=[
            pl.BlockSpec((NC, BLK, HW), lambda i: (0, i, 0)),
            pl.BlockSpec((BLK, D), lambda i: (i, 0)),
            pl.BlockSpec((D, D), lambda i: (0, 0)),
        ],
        out_specs=[
            pl.BlockSpec((NC, BLK, H), lambda i: (0, i, 0)),
            pl.BlockSpec((BLK, 1), lambda i: (i, 0)),
        ],
        out_shape=[
            jax.ShapeDtypeStruct((NC, N, H), jnp.float32),
            jax.ShapeDtypeStruct((N, 1), jnp.float32),
        ],
    )(hist, x, W)

    agg3 = _agg_kernel(z3, src_sc, dst_sc, jnp.zeros((CHUNK, H), jnp.float32))

    y = pl.pallas_call(
        _bn_body,
        grid=(2 * nb,),
        in_specs=[
            pl.BlockSpec((NC, BLK, H), lambda i: (0, i % nb, 0)),
            pl.BlockSpec((BLK, 1), lambda i: (i % nb, 0)),
            pl.BlockSpec((1, D), lambda i: (0, 0)),
            pl.BlockSpec((1, D), lambda i: (0, 0)),
            pl.BlockSpec((1, D), lambda i: (0, 0)),
        ],
        out_specs=pl.BlockSpec((BLK, D), lambda i: (i % nb, 0)),
        out_shape=jax.ShapeDtypeStruct((N, D), jnp.float32),
        scratch_shapes=[
            pltpu.VMEM((1, D), jnp.float32),
            pltpu.VMEM((1, D), jnp.float32),
        ],
    )(agg3, dinv, b.reshape(1, D), gamma.reshape(1, D), beta.reshape(1, D))
    return y
